# baseline (speedup 1.0000x reference)
import jax
import jax.numpy as jnp
from jax import lax
from jax.experimental import pallas as pl
from jax.experimental.pallas import tpu as pltpu
from jax.experimental.pallas import tpu_sc as plsc

B, L = 4096, 200
NC, NS = 2, 16
NW = NC * NS
BW = B // NW
NL = 8
NBLK = L // NL
KCH = 8

PLANE_OFF = [0, 13, 26, 58, 90, 115] + [140 + 101 * j for j in range(8)]
CAT_WORDS = 140 + 101 * 8


def _sc_body(tr_hbm, lb_hbm, cat_hbm,
             o1_hbm, o2_hbm, o3_hbm,
             cat_v, ibuf0, ibuf1, lb0, lb1, o2b0, o2b1, o3b0, o3b1,
             isem0, isem1, osem0, osem1, o1sem):
    c = lax.axis_index("c")
    s = lax.axis_index("s")
    wid = s * NC + c
    b0 = wid * BW
    cols = pl.ds(b0, BW)

    ibufs, lbs = (ibuf0, ibuf1), (lb0, lb1)
    o2bs, o3bs = (o2b0, o2b1), (o3b0, o3b1)
    isems, osems = (isem0, isem1), (osem0, osem1)

    pltpu.sync_copy(cat_hbm, cat_v)

    def fire_in(blk, sl):
        rows = pl.ds(blk * NL, NL)
        pltpu.async_copy(tr_hbm.at[pl.ds(6, 4), rows, cols], ibufs[sl], isems[sl])
        pltpu.async_copy(lb_hbm.at[pl.ds(1, 2), rows, cols], lbs[sl], isems[sl])

    def wait_in(sl):
        rows = pl.ds(0, NL)
        pltpu.make_async_copy(
            tr_hbm.at[pl.ds(6, 4), rows, cols], ibufs[sl], isems[sl]).wait()
        pltpu.make_async_copy(
            lb_hbm.at[pl.ds(1, 2), rows, cols], lbs[sl], isems[sl]).wait()

    def fire_out(blk, sl):
        rows = pl.ds(blk * NL, NL)
        pltpu.async_copy(o2bs[sl], o2_hbm.at[rows, pl.ds(0, 2), cols], osems[sl])
        pltpu.async_copy(o3bs[sl], o3_hbm.at[pl.ds(0, 14), rows, cols], osems[sl])
        pltpu.async_copy(tr_hbm.at[pl.ds(1, 5), rows, cols],
                         o1_hbm.at[pl.ds(0, 5), rows, cols], o1sem)

    def drain_out(sl):
        rows = pl.ds(0, NL)
        pltpu.make_async_copy(
            o2bs[sl], o2_hbm.at[rows, pl.ds(0, 2), cols], osems[sl]).wait()
        pltpu.make_async_copy(
            o3bs[sl], o3_hbm.at[pl.ds(0, 14), rows, cols], osems[sl]).wait()

    def compute(sl):
        ibuf, lb2, o2b, o3b = ibufs[sl], lbs[sl], o2bs[sl], o3bs[sl]

        def row(l, carry):
            for k in range(KCH):
                slk = pl.ds(k * 16, 16)
                ti = ibuf[0, l, slk].astype(jnp.int32)
                mi = ibuf[1, l, slk].astype(jnp.int32)
                di = ibuf[2, l, slk].astype(jnp.int32)
                hi = ibuf[3, l, slk].astype(jnp.int32)
                idx = [mi, mi, di, di, hi, hi] + [ti] * 8
                for j in range(14):
                    o3b[j, l, slk] = plsc.load_gather(cat_v, [idx[j] + PLANE_OFF[j]])
                o2b[l, 0, slk] = lb2[0, l, slk]
                o2b[l, 1, slk] = lb2[1, l, slk]
            return carry

        lax.fori_loop(0, NL, row, 0, unroll=False)

    fire_in(0, 0)
    fire_in(1, 1)

    def ring(g, carry):
        for sl in (0, 1):
            blk = g * 2 + sl
            wait_in(sl)

            @pl.when(g > 0)
            def _():
                drain_out(sl)

            compute(sl)
            fire_out(blk, sl)
            if sl == 0:
                fire_in(blk + 2, sl)
            else:
                @pl.when(g < 11)
                def _():
                    fire_in(blk + 2, sl)
        return carry

    lax.fori_loop(0, 12, ring, 0, unroll=False)

    wait_in(0)
    drain_out(0)
    compute(0)
    fire_out(24, 0)
    drain_out(1)
    drain_out(0)

    def o1drain(i, carry):
        rows = pl.ds(0, NL)
        pltpu.make_async_copy(tr_hbm.at[pl.ds(1, 5), rows, cols],
                              o1_hbm.at[pl.ds(0, 5), rows, cols], o1sem).wait()
        return carry

    lax.fori_loop(0, NBLK, o1drain, 0, unroll=False)


@jax.jit
def _run(tr_t, lb_t, cat):
    f = pl.kernel(
        _sc_body,
        out_type=(
            jax.ShapeDtypeStruct((5, L, B), jnp.float32),
            jax.ShapeDtypeStruct((L, 2, B), jnp.float32),
            jax.ShapeDtypeStruct((14, L, B), jnp.float32),
        ),
        mesh=plsc.VectorSubcoreMesh(
            core_axis_name="c", subcore_axis_name="s",
            num_cores=NC, num_subcores=NS,
        ),
        compiler_params=pltpu.CompilerParams(
            needs_layout_passes=False,
            use_tc_tiling_on_sc=True,
        ),
        scratch_types=[
            pltpu.VMEM((CAT_WORDS,), jnp.float32),
            pltpu.VMEM((4, NL, BW), jnp.float32),
            pltpu.VMEM((4, NL, BW), jnp.float32),
            pltpu.VMEM((2, NL, BW), jnp.float32),
            pltpu.VMEM((2, NL, BW), jnp.float32),
            pltpu.VMEM((NL, 2, BW), jnp.float32),
            pltpu.VMEM((NL, 2, BW), jnp.float32),
            pltpu.VMEM((14, NL, BW), jnp.float32),
            pltpu.VMEM((14, NL, BW), jnp.float32),
            pltpu.SemaphoreType.DMA,
            pltpu.SemaphoreType.DMA,
            pltpu.SemaphoreType.DMA,
            pltpu.SemaphoreType.DMA,
            pltpu.SemaphoreType.DMA,
        ],
    )
    return f(tr_t, lb_t, cat)


def kernel(train, label, month_table, day_table, hour_table, type_table):
    tr_t = jnp.transpose(train, (2, 1, 0))
    lb_t = jnp.transpose(label, (2, 1, 0))
    cat = jnp.concatenate([
        month_table.T.reshape(-1), day_table.T.reshape(-1),
        hour_table.T.reshape(-1), type_table.T.reshape(-1),
    ])
    o1_t, o2_t, o3_t = _run(tr_t, lb_t, cat)
    return (
        jnp.transpose(o1_t, (2, 1, 0)),
        jnp.transpose(o2_t, (2, 0, 1)),
        jnp.transpose(o3_t, (2, 1, 0)),
    )


# speedup vs baseline: 3.8400x; 3.8400x over previous
import jax
import jax.numpy as jnp
from jax import lax
from jax.experimental import pallas as pl
from jax.experimental.pallas import tpu as pltpu
from jax.experimental.pallas import tpu_sc as plsc

B, L = 4096, 200
NC, NS = 2, 16
NW = NC * NS
BW = B // NW
NL = 8
NBLK = L // NL
KCH = 8

PLANE_OFF = [0, 13, 26, 58, 90, 115] + [140 + 101 * j for j in range(8)]
CAT_WORDS = 140 + 101 * 8


def _sc_body(tr_hbm, lb_hbm, cat_hbm,
             o1_hbm, o2_hbm, o3_hbm,
             cat_v, ib0, ib1, lb0, lb1, o1b0, o1b1, o2b0, o2b1, o3b0, o3b1,
             isem0, isem1, osem0, osem1):
    c = lax.axis_index("c")
    s = lax.axis_index("s")
    wid = s * NC + c
    b0 = wid * BW
    cols = pl.ds(b0, BW)

    ibs, lbs = (ib0, ib1), (lb0, lb1)
    o1bs, o2bs, o3bs = (o1b0, o1b1), (o2b0, o2b1), (o3b0, o3b1)
    isems, osems = (isem0, isem1), (osem0, osem1)

    pltpu.sync_copy(cat_hbm, cat_v)

    def fire_in(blk, sl):
        rows = pl.ds(blk * NL, NL)
        pltpu.async_copy(tr_hbm.at[pl.ds(1, 9), rows, cols], ibs[sl], isems[sl])
        pltpu.async_copy(lb_hbm.at[pl.ds(1, 2), rows, cols], lbs[sl], isems[sl])

    def wait_in(sl):
        rows = pl.ds(0, NL)
        pltpu.make_async_copy(
            tr_hbm.at[pl.ds(1, 9), rows, cols], ibs[sl], isems[sl]).wait()
        pltpu.make_async_copy(
            lb_hbm.at[pl.ds(1, 2), rows, cols], lbs[sl], isems[sl]).wait()

    def fire_out(blk, sl):
        rows = pl.ds(blk * NL, NL)
        pltpu.async_copy(o1bs[sl], o1_hbm.at[pl.ds(0, 5), rows, cols], osems[sl])
        pltpu.async_copy(o2bs[sl], o2_hbm.at[rows, pl.ds(0, 2), cols], osems[sl])
        pltpu.async_copy(o3bs[sl], o3_hbm.at[pl.ds(0, 14), rows, cols], osems[sl])

    def drain_out(sl):
        rows = pl.ds(0, NL)
        pltpu.make_async_copy(
            o1bs[sl], o1_hbm.at[pl.ds(0, 5), rows, cols], osems[sl]).wait()
        pltpu.make_async_copy(
            o2bs[sl], o2_hbm.at[rows, pl.ds(0, 2), cols], osems[sl]).wait()
        pltpu.make_async_copy(
            o3bs[sl], o3_hbm.at[pl.ds(0, 14), rows, cols], osems[sl]).wait()

    def compute(sl):
        ib, lb2 = ibs[sl], lbs[sl]
        o1b, o2b, o3b = o1bs[sl], o2bs[sl], o3bs[sl]

        def row(l, carry):
            for k in range(KCH):
                slk = pl.ds(k * 16, 16)
                ti = ib[5, l, slk].astype(jnp.int32)
                mi = ib[6, l, slk].astype(jnp.int32)
                di = ib[7, l, slk].astype(jnp.int32)
                hi = ib[8, l, slk].astype(jnp.int32)
                idx = [mi, mi, di, di, hi, hi] + [ti] * 8
                for j in range(14):
                    o3b[j, l, slk] = plsc.load_gather(cat_v, [idx[j] + PLANE_OFF[j]])
                for p in range(5):
                    o1b[p, l, slk] = ib[p, l, slk]
                o2b[l, 0, slk] = lb2[0, l, slk]
                o2b[l, 1, slk] = lb2[1, l, slk]
            return carry

        lax.fori_loop(0, NL, row, 0, unroll=False)

    fire_in(0, 0)
    fire_in(1, 1)

    def ring(g, carry):
        for sl in (0, 1):
            blk = g * 2 + sl
            wait_in(sl)

            @pl.when(g > 0)
            def _():
                drain_out(sl)

            compute(sl)
            fire_out(blk, sl)
            if sl == 0:
                fire_in(blk + 2, sl)
            else:
                @pl.when(g < 11)
                def _():
                    fire_in(blk + 2, sl)
        return carry

    lax.fori_loop(0, 12, ring, 0, unroll=False)

    wait_in(0)
    drain_out(0)
    compute(0)
    fire_out(24, 0)
    drain_out(1)
    drain_out(0)


@jax.jit
def _run(tr_t, lb_t, cat):
    f = pl.kernel(
        _sc_body,
        out_type=(
            jax.ShapeDtypeStruct((5, L, B), jnp.float32),
            jax.ShapeDtypeStruct((L, 2, B), jnp.float32),
            jax.ShapeDtypeStruct((14, L, B), jnp.float32),
        ),
        mesh=plsc.VectorSubcoreMesh(
            core_axis_name="c", subcore_axis_name="s",
            num_cores=NC, num_subcores=NS,
        ),
        compiler_params=pltpu.CompilerParams(
            needs_layout_passes=False,
            use_tc_tiling_on_sc=True,
        ),
        scratch_types=[
            pltpu.VMEM((CAT_WORDS,), jnp.float32),
            pltpu.VMEM((9, NL, BW), jnp.float32),
            pltpu.VMEM((9, NL, BW), jnp.float32),
            pltpu.VMEM((2, NL, BW), jnp.float32),
            pltpu.VMEM((2, NL, BW), jnp.float32),
            pltpu.VMEM((5, NL, BW), jnp.float32),
            pltpu.VMEM((5, NL, BW), jnp.float32),
            pltpu.VMEM((NL, 2, BW), jnp.float32),
            pltpu.VMEM((NL, 2, BW), jnp.float32),
            pltpu.VMEM((14, NL, BW), jnp.float32),
            pltpu.VMEM((14, NL, BW), jnp.float32),
            pltpu.SemaphoreType.DMA,
            pltpu.SemaphoreType.DMA,
            pltpu.SemaphoreType.DMA,
            pltpu.SemaphoreType.DMA,
        ],
    )
    return f(tr_t, lb_t, cat)


def kernel(train, label, month_table, day_table, hour_table, type_table):
    tr_t = jnp.transpose(train, (2, 1, 0))
    lb_t = jnp.transpose(label, (2, 1, 0))
    cat = jnp.concatenate([
        month_table.T.reshape(-1), day_table.T.reshape(-1),
        hour_table.T.reshape(-1), type_table.T.reshape(-1),
    ])
    o1_t, o2_t, o3_t = _run(tr_t, lb_t, cat)
    return (
        jnp.transpose(o1_t, (2, 1, 0)),
        jnp.transpose(o2_t, (2, 0, 1)),
        jnp.transpose(o3_t, (2, 1, 0)),
    )
